# speedup vs baseline: 12.8485x; 12.8485x over previous
import functools

import jax
import jax.numpy as jnp
from jax import lax
from jax.experimental import pallas as pl


def _fold_kernel(Wm_ref, Wa_ref, bm_ref, WmWa_ref, bmWa_ref):
    WmWa_ref[...] = jnp.dot(Wm_ref[...], Wa_ref[...],
                            preferred_element_type=jnp.float32)
    bmWa_ref[...] = jnp.dot(bm_ref[...], Wa_ref[...],
                            preferred_element_type=jnp.float32)


def _main_kernel(x_blk_ref, x_full_ref, topo_ref, w_ref,
                 WmWa_ref, bmWa_ref, ba_ref, Wg_ref, bg_ref, Wo_ref, bo_ref,
                 out_ref, *, S, K, DIM, T):
    x_b = x_blk_ref[0]
    topo = topo_ref[0]
    w = w_ref[0]

    iota = lax.broadcasted_iota(jnp.int32, (T, S), 1)
    A = jnp.zeros((T, S), dtype=jnp.float32)
    for k in range(K):
        A = A + jnp.where(topo[:, k][:, None] == iota,
                          w[:, k][:, None], 0.0)

    x_full = x_full_ref[0]
    aggx = jnp.dot(A, x_full, preferred_element_type=jnp.float32)
    sumw = jnp.sum(w, axis=1, keepdims=True)

    agg = (jnp.dot(aggx, WmWa_ref[...], preferred_element_type=jnp.float32)
           + sumw * bmWa_ref[...] + ba_ref[...])

    gate_logits = (jnp.dot(x_b, Wg_ref[:DIM], preferred_element_type=jnp.float32)
                   + jnp.dot(agg, Wg_ref[DIM:], preferred_element_type=jnp.float32)
                   + bg_ref[...])
    g = jax.nn.sigmoid(gate_logits)
    upd = x_b + g * (agg - x_b)
    out_ref[0] = jnp.dot(upd, Wo_ref[...],
                         preferred_element_type=jnp.float32) + bo_ref[...]


def kernel(x, topology, weights, Wm, bm, Wa, ba, Wg, bg, Wo, bo):
    B, S, DIM = x.shape
    K = topology.shape[-1]
    T = min(512, S)

    bm2 = bm.reshape(1, DIM)
    ba2 = ba.reshape(1, DIM)
    bg2 = bg.reshape(1, DIM)
    bo2 = bo.reshape(1, DIM)

    WmWa, bmWa = pl.pallas_call(
        _fold_kernel,
        out_shape=(jax.ShapeDtypeStruct((DIM, DIM), jnp.float32),
                   jax.ShapeDtypeStruct((1, DIM), jnp.float32)),
    )(Wm, Wa, bm2)

    grid = (B, S // T)
    out = pl.pallas_call(
        functools.partial(_main_kernel, S=S, K=K, DIM=DIM, T=T),
        grid=grid,
        in_specs=[
            pl.BlockSpec((1, T, DIM), lambda b, t: (b, t, 0)),
            pl.BlockSpec((1, S, DIM), lambda b, t: (b, 0, 0)),
            pl.BlockSpec((1, T, K), lambda b, t: (b, t, 0)),
            pl.BlockSpec((1, T, K), lambda b, t: (b, t, 0)),
            pl.BlockSpec((DIM, DIM), lambda b, t: (0, 0)),
            pl.BlockSpec((1, DIM), lambda b, t: (0, 0)),
            pl.BlockSpec((1, DIM), lambda b, t: (0, 0)),
            pl.BlockSpec((2 * DIM, DIM), lambda b, t: (0, 0)),
            pl.BlockSpec((1, DIM), lambda b, t: (0, 0)),
            pl.BlockSpec((DIM, DIM), lambda b, t: (0, 0)),
            pl.BlockSpec((1, DIM), lambda b, t: (0, 0)),
        ],
        out_specs=pl.BlockSpec((1, T, DIM), lambda b, t: (b, t, 0)),
        out_shape=jax.ShapeDtypeStruct((B, S, DIM), jnp.float32),
    )(x, x, topology, weights, WmWa, bmWa, ba2, Wg, bg2, Wo, bo2)
    return out
